# baseline (speedup 1.0000x reference)
import functools

import jax
import jax.numpy as jnp
from jax import lax
from jax.experimental import pallas as pl
from jax.experimental.pallas import tpu as pltpu
from jax.experimental.pallas import tpu_sc as plsc

CHUNK = 1600


@functools.lru_cache(maxsize=None)
def _make_gather(B, V, D):
    info = plsc.get_sparse_core_info()
    NC, NS = info.num_cores, info.num_subcores
    NW = NC * NS
    assert B % (NW * CHUNK) == 0
    b_per_w = B // NW
    n_chunk = b_per_w // CHUNK
    mesh = plsc.VectorSubcoreMesh(core_axis_name="c", subcore_axis_name="s")

    @functools.partial(
        pl.kernel,
        mesh=mesh,
        compiler_params=pltpu.CompilerParams(use_tc_tiling_on_sc=False),
        out_type=jax.ShapeDtypeStruct((B, D), jnp.float32),
        scratch_types=[
            pltpu.VMEM((b_per_w,), jnp.int32),
            pltpu.VMEM((2, CHUNK, D), jnp.float32),
            pltpu.SemaphoreType.DMA((2,)),
        ],
    )
    def gather_kernel(idx_hbm, table_hbm, out_hbm, idx_v, rows_v, gsem):
        wid = lax.axis_index("s") * NC + lax.axis_index("c")
        base = wid * b_per_w
        pltpu.sync_copy(idx_hbm.at[pl.ds(base, b_per_w)], idx_v)

        def fire(b, slot):
            pltpu.async_copy(
                table_hbm.at[idx_v.at[pl.ds(b * CHUNK, CHUNK)]],
                rows_v.at[slot],
                gsem.at[slot],
            )

        def drain_and_write(b, slot):
            pltpu.make_async_copy(
                table_hbm.at[idx_v.at[pl.ds(b * CHUNK, CHUNK)]],
                rows_v.at[slot],
                gsem.at[slot],
            ).wait()
            out_base = pl.multiple_of(base + b * CHUNK, CHUNK)
            pltpu.sync_copy(rows_v.at[slot], out_hbm.at[pl.ds(out_base, CHUNK)])

        fire(0, 0)

        def step(b, _):
            fire(b + 1, lax.rem(b + 1, 2))
            drain_and_write(b, lax.rem(b, 2))
            return ()

        lax.fori_loop(0, n_chunk - 1, step, (), unroll=False)
        drain_and_write(n_chunk - 1, (n_chunk - 1) % 2)

    return gather_kernel


def kernel(input_, table_ids):
    B0, H = input_.shape
    V, D = table_ids.shape
    B = B0 * H
    idx = input_.astype(jnp.int32).reshape(B)
    out = _make_gather(B, V, D)(idx, table_ids)
    return out.reshape(B0, H, D)


# speedup vs baseline: 1.3717x; 1.3717x over previous
import functools

import jax
import jax.numpy as jnp
from jax import lax
from jax.experimental import pallas as pl
from jax.experimental.pallas import tpu as pltpu
from jax.experimental.pallas import tpu_sc as plsc

BLK = 128


@functools.lru_cache(maxsize=None)
def _make_gather(B0, H, V, D):
    info = plsc.get_sparse_core_info()
    NC, NS, L = info.num_cores, info.num_subcores, info.num_lanes
    NW = NC * NS
    n_blk = B0 // BLK
    blk_per_w = n_blk // NW
    D_HI = D // 8
    mesh = plsc.VectorSubcoreMesh(core_axis_name="c", subcore_axis_name="s")

    @functools.partial(
        pl.kernel,
        mesh=mesh,
        compiler_params=pltpu.CompilerParams(
            use_tc_tiling_on_sc=False, needs_layout_passes=False),
        out_type=jax.ShapeDtypeStruct((H, D_HI, n_blk, 8, BLK), jnp.float32),
        scratch_types=[
            pltpu.VMEM((BLK * H,), jnp.int32),
            pltpu.VMEM((H * BLK,), jnp.int32),
            pltpu.VMEM((2, BLK, D), jnp.float32),
            pltpu.VMEM((D_HI, 8, BLK), jnp.float32),
            pltpu.SemaphoreType.DMA((2,)),
        ],
    )
    def gather_kernel(idx_hbm, table_hbm, out_hbm, idx_v, idxt_v, rows_v,
                      t_v, gsem):
        wid = lax.axis_index("s") * NC + lax.axis_index("c")

        def fire(h, slot):
            pltpu.async_copy(
                table_hbm.at[idxt_v.at[pl.ds(h * BLK, BLK)]],
                rows_v.at[slot],
                gsem.at[slot],
            )

        def wait(h, slot):
            pltpu.make_async_copy(
                table_hbm.at[idxt_v.at[pl.ds(h * BLK, BLK)]],
                rows_v.at[slot],
                gsem.at[slot],
            ).wait()

        def do_block(blk, _):
            bhi = wid * blk_per_w + blk
            pltpu.sync_copy(idx_hbm.at[pl.ds(bhi * (BLK * H), BLK * H)],
                            idx_v)

            def idx_t(g, _):
                h = g // (BLK // L)
                k = g - h * (BLK // L)
                iota_h = lax.iota(jnp.int32, L) * H
                src = plsc.load_gather(idx_v, [iota_h + (k * (L * H) + h)])
                idxt_v[pl.ds(g * L, L)] = src
                return ()

            lax.fori_loop(0, H * (BLK // L), idx_t, (), unroll=False)

            fire(0, 0)

            def do_h(h, _):
                slot = lax.rem(h, 2)

                @pl.when(h + 1 < H)
                def _():
                    fire(h + 1, lax.rem(h + 1, 2))

                wait(h, slot)

                def row_t(q, _):
                    d = q // (BLK // L)
                    k = q - d * (BLK // L)
                    iota = lax.iota(jnp.int32, L)
                    src = plsc.load_gather(
                        rows_v.at[slot], [iota + k * L, iota * 0 + d])
                    t_v[d // 8, lax.rem(d, 8), pl.ds(k * L, L)] = src
                    return ()

                lax.fori_loop(0, D * (BLK // L), row_t, (), unroll=False)

                for d_hi in range(D_HI):
                    pltpu.sync_copy(t_v.at[d_hi], out_hbm.at[h, d_hi, bhi])
                return ()

            lax.fori_loop(0, H, do_h, (), unroll=False)
            return ()

        lax.fori_loop(0, blk_per_w, do_block, (), unroll=False)

    return gather_kernel


def kernel(input_, table_ids):
    B0, H = input_.shape
    V, D = table_ids.shape
    idx = input_.astype(jnp.int32).reshape(B0 * H)
    out5 = _make_gather(B0, H, V, D)(idx, table_ids)
    return out5.transpose(2, 4, 0, 1, 3).reshape(B0, H, D)


# speedup vs baseline: 1.3736x; 1.0014x over previous
import functools

import jax
import jax.numpy as jnp
from jax import lax
from jax.experimental import pallas as pl
from jax.experimental.pallas import tpu as pltpu
from jax.experimental.pallas import tpu_sc as plsc

BLK = 128


@functools.lru_cache(maxsize=None)
def _make_gather(B0, H, V, D):
    info = plsc.get_sparse_core_info()
    NC, NS, L = info.num_cores, info.num_subcores, info.num_lanes
    NW = NC * NS
    n_blk = B0 // BLK
    blk_per_w = n_blk // NW
    D_HI = D // 8
    mesh = plsc.VectorSubcoreMesh(core_axis_name="c", subcore_axis_name="s")

    @functools.partial(
        pl.kernel,
        mesh=mesh,
        compiler_params=pltpu.CompilerParams(
            use_tc_tiling_on_sc=False, needs_layout_passes=False),
        out_type=jax.ShapeDtypeStruct((H, D_HI, n_blk, 8, BLK), jnp.float32),
        scratch_types=[
            pltpu.VMEM((BLK * H,), jnp.int32),
            pltpu.VMEM((H * BLK,), jnp.int32),
            pltpu.VMEM((2, BLK, D), jnp.float32),
            pltpu.VMEM((D_HI, 8, BLK), jnp.float32),
            pltpu.SemaphoreType.DMA((2,)),
        ],
    )
    def gather_kernel(idx_hbm, table_hbm, out_hbm, idx_v, idxt_v, rows_v,
                      t_v, gsem):
        wid = lax.axis_index("s") * NC + lax.axis_index("c")

        def fire(h, slot):
            pltpu.async_copy(
                table_hbm.at[idxt_v.at[pl.ds(h * BLK, BLK)]],
                rows_v.at[slot],
                gsem.at[slot],
            )

        def wait(h, slot):
            pltpu.make_async_copy(
                table_hbm.at[idxt_v.at[pl.ds(h * BLK, BLK)]],
                rows_v.at[slot],
                gsem.at[slot],
            ).wait()

        def do_block(blk, _):
            bhi = wid * blk_per_w + blk
            pltpu.sync_copy(idx_hbm.at[pl.ds(bhi * (BLK * H), BLK * H)],
                            idx_v)

            def idx_t(g, _):
                h = g // (BLK // L)
                k = g - h * (BLK // L)
                iota_h = lax.iota(jnp.int32, L) * H
                src = plsc.load_gather(idx_v, [iota_h + (k * (L * H) + h)])
                idxt_v[pl.ds(g * L, L)] = src
                return ()

            lax.fori_loop(0, H * (BLK // L), idx_t, (), unroll=8)

            fire(0, 0)

            def do_h(h, _):
                slot = lax.rem(h, 2)

                @pl.when(h + 1 < H)
                def _():
                    fire(h + 1, lax.rem(h + 1, 2))

                wait(h, slot)

                def row_t(q, _):
                    d = q // (BLK // L)
                    k = q - d * (BLK // L)
                    iota = lax.iota(jnp.int32, L)
                    src = plsc.load_gather(
                        rows_v.at[slot], [iota + k * L, iota * 0 + d])
                    t_v[d // 8, lax.rem(d, 8), pl.ds(k * L, L)] = src
                    return ()

                lax.fori_loop(0, D * (BLK // L), row_t, (), unroll=8)

                for d_hi in range(D_HI):
                    pltpu.sync_copy(t_v.at[d_hi], out_hbm.at[h, d_hi, bhi])
                return ()

            lax.fori_loop(0, H, do_h, (), unroll=False)
            return ()

        lax.fori_loop(0, blk_per_w, do_block, (), unroll=False)

    return gather_kernel


def kernel(input_, table_ids):
    B0, H = input_.shape
    V, D = table_ids.shape
    idx = input_.astype(jnp.int32).reshape(B0 * H)
    out5 = _make_gather(B0, H, V, D)(idx, table_ids)
    return out5.transpose(2, 4, 0, 1, 3).reshape(B0, H, D)


# speedup vs baseline: 2.1277x; 1.5490x over previous
import functools

import jax
import jax.numpy as jnp
from jax import lax
from jax.experimental import pallas as pl
from jax.experimental.pallas import tpu as pltpu
from jax.experimental.pallas import tpu_sc as plsc

BLK = 128


@functools.lru_cache(maxsize=None)
def _make_gather(B0, H, V, D):
    info = plsc.get_sparse_core_info()
    NC, NS, L = info.num_cores, info.num_subcores, info.num_lanes
    NW = NC * NS
    n_blk = B0 // BLK
    blk_per_w = n_blk // NW
    D_HI = D // 8
    mesh = plsc.VectorSubcoreMesh(core_axis_name="c", subcore_axis_name="s")

    @functools.partial(
        pl.kernel,
        mesh=mesh,
        compiler_params=pltpu.CompilerParams(
            use_tc_tiling_on_sc=False, needs_layout_passes=False),
        out_type=jax.ShapeDtypeStruct((H, D_HI, n_blk, 8, BLK), jnp.float32),
        scratch_types=[
            pltpu.VMEM((BLK * H,), jnp.int32),
            pltpu.VMEM((H * BLK,), jnp.int32),
            pltpu.VMEM((2, BLK, D), jnp.float32),
            pltpu.VMEM((D, BLK + 8), jnp.float32),
            pltpu.SemaphoreType.DMA((2,)),
        ],
    )
    def gather_kernel(idx_hbm, table_hbm, out_hbm, idx_v, idxt_v, rows_v,
                      t_v, gsem):
        wid = lax.axis_index("s") * NC + lax.axis_index("c")

        def fire(h, slot):
            pltpu.async_copy(
                table_hbm.at[idxt_v.at[pl.ds(h * BLK, BLK)]],
                rows_v.at[slot],
                gsem.at[slot],
            )

        def wait(h, slot):
            pltpu.make_async_copy(
                table_hbm.at[idxt_v.at[pl.ds(h * BLK, BLK)]],
                rows_v.at[slot],
                gsem.at[slot],
            ).wait()

        def do_block(blk, _):
            bhi = wid * blk_per_w + blk
            pltpu.sync_copy(idx_hbm.at[pl.ds(bhi * (BLK * H), BLK * H)],
                            idx_v)

            def idx_t(g, _):
                h = g // (BLK // L)
                k = g - h * (BLK // L)
                iota_h = lax.iota(jnp.int32, L) * H
                src = plsc.load_gather(idx_v, [iota_h + (k * (L * H) + h)])
                idxt_v[pl.ds(g * L, L)] = src
                return ()

            lax.fori_loop(0, H * (BLK // L), idx_t, (), unroll=8)

            fire(0, 0)

            def do_h(h, _):
                slot = lax.rem(h, 2)

                @pl.when(h + 1 < H)
                def _():
                    fire(h + 1, lax.rem(h + 1, 2))

                wait(h, slot)

                def row_t(q, _):
                    b = q // (D // L)
                    d0 = lax.rem(q, D // L) * L
                    iota = lax.iota(jnp.int32, L)
                    src = rows_v[slot, b, pl.ds(d0, L)]
                    plsc.store_scatter(t_v, [iota + d0, iota * 0 + b], src)
                    return ()

                lax.fori_loop(0, BLK * (D // L), row_t, (), unroll=8)

                for d_hi in range(D_HI):
                    pltpu.sync_copy(t_v.at[pl.ds(8 * d_hi, 8), pl.ds(0, BLK)],
                                    out_hbm.at[h, d_hi, bhi])
                return ()

            lax.fori_loop(0, H, do_h, (), unroll=False)
            return ()

        lax.fori_loop(0, blk_per_w, do_block, (), unroll=False)

    return gather_kernel


def kernel(input_, table_ids):
    B0, H = input_.shape
    V, D = table_ids.shape
    idx = input_.astype(jnp.int32).reshape(B0 * H)
    out5 = _make_gather(B0, H, V, D)(idx, table_ids)
    return out5.transpose(2, 4, 0, 1, 3).reshape(B0, H, D)
